# baseline (speedup 1.0000x reference)
import functools

import jax
import jax.numpy as jnp
from jax.experimental import pallas as pl
from jax.experimental.pallas import tpu as pltpu
from jax.experimental.pallas import tpu_sc as plsc

_GATHER_WINDOW = 128
_EDGE_BLOCK = 8000
_CHUNK_SIZES = (16000, 112000, 112000, 80000)


def _project_body(nfeat_ref, w1_ref, out_ref):
    out_ref[...] = jnp.dot(nfeat_ref[...], w1_ref[...],
                           preferred_element_type=jnp.float32)


def _project(nfeat, w1):
    n, d = nfeat.shape
    h = w1.shape[1]
    return pl.pallas_call(
        _project_body,
        grid=(2,),
        in_specs=[
            pl.BlockSpec((n, d), lambda j: (0, 0)),
            pl.BlockSpec((d, h), lambda j: (j + 1, 0)),
        ],
        out_specs=pl.BlockSpec((n, h), lambda j: (j, 0)),
        out_shape=jax.ShapeDtypeStruct((2 * n, h), jnp.float32),
    )(nfeat, w1)


_LOADER_ROWS = 1000


def _sc_gather(table, idx_all, start, size):
    n = table.shape[0] // 2
    h = table.shape[1]
    assert size % _GATHER_WINDOW == 0 and start % _GATHER_WINDOW == 0
    assert n % _LOADER_ROWS == 0 and _LOADER_ROWS % 8 == 0
    n_loaders = n // _LOADER_ROWS
    start_blk = start // _GATHER_WINDOW
    mesh = plsc.VectorSubcoreMesh(core_axis_name="c", subcore_axis_name="s")

    @functools.partial(
        pl.kernel,
        out_type=jax.ShapeDtypeStruct((2 * size, h), jnp.float32),
        mesh=mesh,
        scratch_types=[pltpu.VMEM_SHARED((n, h), jnp.float32)],
    )
    def gather_kernel(table_hbm, idx_hbm, out_hbm, spmem):
        c = jax.lax.axis_index("c")
        s = jax.lax.axis_index("s")

        @pl.when(s < n_loaders)
        def _load():
            pltpu.sync_copy(
                table_hbm.at[pl.ds(c * n + s * _LOADER_ROWS, _LOADER_ROWS)],
                spmem.at[pl.ds(s * _LOADER_ROWS, _LOADER_ROWS)])

        plsc.subcore_barrier()

        def body(i_vmem, o_vmem):
            pltpu.sync_copy(spmem.at[i_vmem.at[0]], o_vmem)

        pltpu.emit_pipeline(
            body,
            grid=(size // _GATHER_WINDOW,),
            in_specs=[pl.BlockSpec((1, _GATHER_WINDOW),
                                   lambda i: (0, i + start_blk))],
            out_specs=[pl.BlockSpec((_GATHER_WINDOW, h), lambda i: (i, 0))],
            core_axis_name=("s",),
            dimension_semantics=(pltpu.PARALLEL,),
        )(idx_hbm.at[c], out_hbm.at[pl.ds(c * size, size)])

    return gather_kernel(table, idx_all)


def _mlp_body(ef_ref, gs_ref, gd_ref, w1_ref, b1_ref, w2_ref, b2_ref,
              lg_ref, lb_ref, out_ref):
    ef = ef_ref[...]
    h = jnp.dot(ef.astype(jnp.bfloat16), w1_ref[...].astype(jnp.bfloat16),
                preferred_element_type=jnp.float32)
    h = h + gs_ref[...] + gd_ref[...] + b1_ref[...]
    h = h * jax.lax.logistic(h)
    h = jnp.dot(h.astype(jnp.bfloat16), w2_ref[...].astype(jnp.bfloat16),
                preferred_element_type=jnp.float32)
    h = h + b2_ref[...]
    mu = jnp.mean(h, axis=-1, keepdims=True)
    var = jnp.mean((h - mu) * (h - mu), axis=-1, keepdims=True)
    h = (h - mu) * jax.lax.rsqrt(var + 1e-5) * lg_ref[...] + lb_ref[...]
    out_ref[...] = h + ef


def _mlp_chunk(base, nblk_chunk, efeat, gs, gd, w1, b1, w2, b2,
               ln_g, ln_b, prev):
    e, d = efeat.shape
    h = w1.shape[1]
    body = _mlp_body if prev is None else (
        lambda ef, gs_, gd_, w1_, b1_, w2_, b2_, lg, lb, _prev, out:
        _mlp_body(ef, gs_, gd_, w1_, b1_, w2_, b2_, lg, lb, out))
    nblk = nblk_chunk
    in_specs = [
        pl.BlockSpec((_EDGE_BLOCK, d), lambda i: (i + base, 0)),
        pl.BlockSpec((_EDGE_BLOCK, h), lambda i: (i, 0)),
        pl.BlockSpec((_EDGE_BLOCK, h), lambda i: (i + nblk, 0)),
        pl.BlockSpec((d, h), lambda i: (0, 0)),
        pl.BlockSpec((1, h), lambda i: (0, 0)),
        pl.BlockSpec((h, d), lambda i: (0, 0)),
        pl.BlockSpec((1, d), lambda i: (0, 0)),
        pl.BlockSpec((1, d), lambda i: (0, 0)),
        pl.BlockSpec((1, d), lambda i: (0, 0)),
    ]
    args = [efeat, gs, gd, w1, b1.reshape(1, h), w2,
            b2.reshape(1, d), ln_g.reshape(1, d), ln_b.reshape(1, d)]
    aliases = {}
    if prev is not None:
        in_specs.append(pl.BlockSpec(memory_space=pl.ANY))
        args.append(prev)
        aliases = {9: 0}
    return pl.pallas_call(
        body,
        grid=(nblk_chunk,),
        in_specs=in_specs,
        out_specs=pl.BlockSpec((_EDGE_BLOCK, d), lambda i: (i + base, 0)),
        out_shape=jax.ShapeDtypeStruct((e, d), jnp.float32),
        input_output_aliases=aliases,
    )(*args)


def kernel(efeat, nfeat, edge_index, w1, b1, w2, b2, ln_g, ln_b):
    e, d = efeat.shape
    table = _project(nfeat, w1)

    sizes = [s for s in _CHUNK_SIZES if s]
    assert sum(sizes) == e, (sizes, e)

    idx_all = edge_index.reshape(2, 1, e)
    gathers = []
    start = 0
    for sz in sizes:
        gathers.append(_sc_gather(table, idx_all, start, sz))
        start += sz

    out = None
    start = 0
    for c, sz in enumerate(sizes):
        out = _mlp_chunk(start // _EDGE_BLOCK, sz // _EDGE_BLOCK, efeat,
                         gathers[c], gathers[c], w1, b1, w2, b2,
                         ln_g, ln_b, out)
        start += sz
    return (out, nfeat)


# speedup vs baseline: 1.0169x; 1.0169x over previous
import functools

import jax
import jax.numpy as jnp
from jax.experimental import pallas as pl
from jax.experimental.pallas import tpu as pltpu
from jax.experimental.pallas import tpu_sc as plsc

_GATHER_WINDOW = 128
_EDGE_BLOCK = 8000
_CHUNK_SIZES = (80000, 80000, 80000, 80000)


def _project_body(nfeat_ref, w1_ref, out_ref):
    out_ref[...] = jnp.dot(nfeat_ref[...], w1_ref[...],
                           preferred_element_type=jnp.float32)


def _project(nfeat, w1):
    n, d = nfeat.shape
    h = w1.shape[1]
    return pl.pallas_call(
        _project_body,
        grid=(2,),
        in_specs=[
            pl.BlockSpec((n, d), lambda j: (0, 0)),
            pl.BlockSpec((d, h), lambda j: (j + 1, 0)),
        ],
        out_specs=pl.BlockSpec((n, h), lambda j: (j, 0)),
        out_shape=jax.ShapeDtypeStruct((2 * n, h), jnp.float32),
    )(nfeat, w1)


_LOADER_ROWS = 1000


def _sc_gather(table, idx_all, start, size):
    n = table.shape[0] // 2
    h = table.shape[1]
    assert size % _GATHER_WINDOW == 0 and start % _GATHER_WINDOW == 0
    assert n % _LOADER_ROWS == 0 and _LOADER_ROWS % 8 == 0
    n_loaders = n // _LOADER_ROWS
    start_blk = start // _GATHER_WINDOW
    mesh = plsc.VectorSubcoreMesh(core_axis_name="c", subcore_axis_name="s")

    @functools.partial(
        pl.kernel,
        out_type=jax.ShapeDtypeStruct((2 * size, h), jnp.float32),
        mesh=mesh,
        scratch_types=[pltpu.VMEM_SHARED((n, h), jnp.float32)],
    )
    def gather_kernel(table_hbm, idx_hbm, out_hbm, spmem):
        c = jax.lax.axis_index("c")
        s = jax.lax.axis_index("s")

        @pl.when(s < n_loaders)
        def _load():
            pltpu.sync_copy(
                table_hbm.at[pl.ds(c * n + s * _LOADER_ROWS, _LOADER_ROWS)],
                spmem.at[pl.ds(s * _LOADER_ROWS, _LOADER_ROWS)])

        plsc.subcore_barrier()

        def body(i_vmem, o_vmem):
            pltpu.sync_copy(spmem.at[i_vmem.at[0]], o_vmem)

        pltpu.emit_pipeline(
            body,
            grid=(size // _GATHER_WINDOW,),
            in_specs=[pl.BlockSpec((1, _GATHER_WINDOW),
                                   lambda i: (0, i + start_blk))],
            out_specs=[pl.BlockSpec((_GATHER_WINDOW, h), lambda i: (i, 0))],
            core_axis_name=("s",),
            dimension_semantics=(pltpu.PARALLEL,),
        )(idx_hbm.at[c], out_hbm.at[pl.ds(c * size, size)])

    return gather_kernel(table, idx_all)


def _mlp_body(ef_ref, gs_ref, gd_ref, w1_ref, b1_ref, w2_ref, b2_ref,
              lg_ref, lb_ref, out_ref):
    ef = ef_ref[...]
    h = jnp.dot(ef.astype(jnp.bfloat16), w1_ref[...].astype(jnp.bfloat16),
                preferred_element_type=jnp.float32)
    h = h + gs_ref[...] + gd_ref[...] + b1_ref[...]
    h = h * jax.lax.logistic(h)
    h = jnp.dot(h.astype(jnp.bfloat16), w2_ref[...].astype(jnp.bfloat16),
                preferred_element_type=jnp.float32)
    h = h + b2_ref[...]
    mu = jnp.mean(h, axis=-1, keepdims=True)
    var = jnp.mean((h - mu) * (h - mu), axis=-1, keepdims=True)
    h = (h - mu) * jax.lax.rsqrt(var + 1e-5) * lg_ref[...] + lb_ref[...]
    out_ref[...] = h + ef


def _mlp_chunk(base, nblk_chunk, efeat, gs, gd, w1, b1, w2, b2,
               ln_g, ln_b, prev):
    e, d = efeat.shape
    h = w1.shape[1]
    body = _mlp_body if prev is None else (
        lambda ef, gs_, gd_, w1_, b1_, w2_, b2_, lg, lb, _prev, out:
        _mlp_body(ef, gs_, gd_, w1_, b1_, w2_, b2_, lg, lb, out))
    nblk = nblk_chunk
    in_specs = [
        pl.BlockSpec((_EDGE_BLOCK, d), lambda i: (i + base, 0)),
        pl.BlockSpec((_EDGE_BLOCK, h), lambda i: (i, 0)),
        pl.BlockSpec((_EDGE_BLOCK, h), lambda i: (i + nblk, 0)),
        pl.BlockSpec((d, h), lambda i: (0, 0)),
        pl.BlockSpec((1, h), lambda i: (0, 0)),
        pl.BlockSpec((h, d), lambda i: (0, 0)),
        pl.BlockSpec((1, d), lambda i: (0, 0)),
        pl.BlockSpec((1, d), lambda i: (0, 0)),
        pl.BlockSpec((1, d), lambda i: (0, 0)),
    ]
    args = [efeat, gs, gd, w1, b1.reshape(1, h), w2,
            b2.reshape(1, d), ln_g.reshape(1, d), ln_b.reshape(1, d)]
    aliases = {}
    if prev is not None:
        in_specs.append(pl.BlockSpec(memory_space=pl.ANY))
        args.append(prev)
        aliases = {9: 0}
    return pl.pallas_call(
        body,
        grid=(nblk_chunk,),
        in_specs=in_specs,
        out_specs=pl.BlockSpec((_EDGE_BLOCK, d), lambda i: (i + base, 0)),
        out_shape=jax.ShapeDtypeStruct((e, d), jnp.float32),
        input_output_aliases=aliases,
    )(*args)


def kernel(efeat, nfeat, edge_index, w1, b1, w2, b2, ln_g, ln_b):
    e, d = efeat.shape
    table = _project(nfeat, w1)

    sizes = [s for s in _CHUNK_SIZES if s]
    assert sum(sizes) == e, (sizes, e)

    idx_all = edge_index.reshape(2, 1, e)
    gathers = []
    start = 0
    for sz in sizes:
        gathers.append(_sc_gather(table, idx_all, start, sz))
        start += sz

    out = None
    start = 0
    for c, sz in enumerate(sizes):
        out = _mlp_chunk(start // _EDGE_BLOCK, sz // _EDGE_BLOCK, efeat,
                         gathers[c], gathers[c], w1, b1, w2, b2,
                         ln_g, ln_b, out)
        start += sz
    return (out, nfeat)


# speedup vs baseline: 1.0226x; 1.0055x over previous
import functools

import jax
import jax.numpy as jnp
from jax.experimental import pallas as pl
from jax.experimental.pallas import tpu as pltpu
from jax.experimental.pallas import tpu_sc as plsc

_GATHER_WINDOW = 128
_EDGE_BLOCK = 10000
_CHUNK_SIZES = (80000, 80000, 80000, 80000)


def _project_body(nfeat_ref, w1_ref, out_ref):
    out_ref[...] = jnp.dot(nfeat_ref[...], w1_ref[...],
                           preferred_element_type=jnp.float32)


def _project(nfeat, w1):
    n, d = nfeat.shape
    h = w1.shape[1]
    return pl.pallas_call(
        _project_body,
        grid=(2,),
        in_specs=[
            pl.BlockSpec((n, d), lambda j: (0, 0)),
            pl.BlockSpec((d, h), lambda j: (j + 1, 0)),
        ],
        out_specs=pl.BlockSpec((n, h), lambda j: (j, 0)),
        out_shape=jax.ShapeDtypeStruct((2 * n, h), jnp.float32),
    )(nfeat, w1)


_LOADER_ROWS = 1000


def _sc_gather(table, idx_all, start, size):
    n = table.shape[0] // 2
    h = table.shape[1]
    assert size % _GATHER_WINDOW == 0 and start % _GATHER_WINDOW == 0
    assert n % _LOADER_ROWS == 0 and _LOADER_ROWS % 8 == 0
    n_loaders = n // _LOADER_ROWS
    start_blk = start // _GATHER_WINDOW
    mesh = plsc.VectorSubcoreMesh(core_axis_name="c", subcore_axis_name="s")

    @functools.partial(
        pl.kernel,
        out_type=jax.ShapeDtypeStruct((2 * size, h), jnp.float32),
        mesh=mesh,
        scratch_types=[pltpu.VMEM_SHARED((n, h), jnp.float32)],
    )
    def gather_kernel(table_hbm, idx_hbm, out_hbm, spmem):
        c = jax.lax.axis_index("c")
        s = jax.lax.axis_index("s")

        @pl.when(s < n_loaders)
        def _load():
            pltpu.sync_copy(
                table_hbm.at[pl.ds(c * n + s * _LOADER_ROWS, _LOADER_ROWS)],
                spmem.at[pl.ds(s * _LOADER_ROWS, _LOADER_ROWS)])

        plsc.subcore_barrier()

        def body(i_vmem, o_vmem):
            pltpu.sync_copy(spmem.at[i_vmem.at[0]], o_vmem)

        pltpu.emit_pipeline(
            body,
            grid=(size // _GATHER_WINDOW,),
            in_specs=[pl.BlockSpec((1, _GATHER_WINDOW),
                                   lambda i: (0, i + start_blk))],
            out_specs=[pl.BlockSpec((_GATHER_WINDOW, h), lambda i: (i, 0))],
            core_axis_name=("s",),
            dimension_semantics=(pltpu.PARALLEL,),
        )(idx_hbm.at[c], out_hbm.at[pl.ds(c * size, size)])

    return gather_kernel(table, idx_all)


def _mlp_body(ef_ref, gs_ref, gd_ref, w1_ref, b1_ref, w2_ref, b2_ref,
              lg_ref, lb_ref, out_ref):
    ef = ef_ref[...]
    h = jnp.dot(ef.astype(jnp.bfloat16), w1_ref[...].astype(jnp.bfloat16),
                preferred_element_type=jnp.float32)
    h = h + gs_ref[...] + gd_ref[...] + b1_ref[...]
    h = h * jax.lax.logistic(h)
    h = jnp.dot(h.astype(jnp.bfloat16), w2_ref[...].astype(jnp.bfloat16),
                preferred_element_type=jnp.float32)
    h = h + b2_ref[...]
    mu = jnp.mean(h, axis=-1, keepdims=True)
    var = jnp.mean((h - mu) * (h - mu), axis=-1, keepdims=True)
    h = (h - mu) * jax.lax.rsqrt(var + 1e-5) * lg_ref[...] + lb_ref[...]
    out_ref[...] = h + ef


def _mlp_chunk(base, nblk_chunk, efeat, gs, gd, w1, b1, w2, b2,
               ln_g, ln_b, prev):
    e, d = efeat.shape
    h = w1.shape[1]
    body = _mlp_body if prev is None else (
        lambda ef, gs_, gd_, w1_, b1_, w2_, b2_, lg, lb, _prev, out:
        _mlp_body(ef, gs_, gd_, w1_, b1_, w2_, b2_, lg, lb, out))
    nblk = nblk_chunk
    in_specs = [
        pl.BlockSpec((_EDGE_BLOCK, d), lambda i: (i + base, 0)),
        pl.BlockSpec((_EDGE_BLOCK, h), lambda i: (i, 0)),
        pl.BlockSpec((_EDGE_BLOCK, h), lambda i: (i + nblk, 0)),
        pl.BlockSpec((d, h), lambda i: (0, 0)),
        pl.BlockSpec((1, h), lambda i: (0, 0)),
        pl.BlockSpec((h, d), lambda i: (0, 0)),
        pl.BlockSpec((1, d), lambda i: (0, 0)),
        pl.BlockSpec((1, d), lambda i: (0, 0)),
        pl.BlockSpec((1, d), lambda i: (0, 0)),
    ]
    args = [efeat, gs, gd, w1, b1.reshape(1, h), w2,
            b2.reshape(1, d), ln_g.reshape(1, d), ln_b.reshape(1, d)]
    aliases = {}
    if prev is not None:
        in_specs.append(pl.BlockSpec(memory_space=pl.ANY))
        args.append(prev)
        aliases = {9: 0}
    return pl.pallas_call(
        body,
        grid=(nblk_chunk,),
        in_specs=in_specs,
        out_specs=pl.BlockSpec((_EDGE_BLOCK, d), lambda i: (i + base, 0)),
        out_shape=jax.ShapeDtypeStruct((e, d), jnp.float32),
        input_output_aliases=aliases,
    )(*args)


def kernel(efeat, nfeat, edge_index, w1, b1, w2, b2, ln_g, ln_b):
    e, d = efeat.shape
    table = _project(nfeat, w1)

    sizes = [s for s in _CHUNK_SIZES if s]
    assert sum(sizes) == e, (sizes, e)

    idx_all = edge_index.reshape(2, 1, e)
    gathers = []
    start = 0
    for sz in sizes:
        gathers.append(_sc_gather(table, idx_all, start, sz))
        start += sz

    out = None
    start = 0
    for c, sz in enumerate(sizes):
        out = _mlp_chunk(start // _EDGE_BLOCK, sz // _EDGE_BLOCK, efeat,
                         gathers[c], gathers[c], w1, b1, w2, b2,
                         ln_g, ln_b, out)
        start += sz
    return (out, nfeat)
